# baseline (speedup 1.0000x reference)
import functools

import jax
import jax.numpy as jnp
from jax import lax
from jax.experimental import pallas as pl
from jax.experimental.pallas import tpu as pltpu
from jax.experimental.pallas import tpu_sc as plsc

NUM_EXPERTS = 64
NUM_KEPT = 48
NEG_INF = float("-inf")

D_MODEL = 1024
D_HALF = D_MODEL // 2
D_FF = 384
TWO_F = 2 * D_FF
N_TOK = 4096
TB_R = 512
NB_R = N_TOK // TB_R
TM = 256
G_TILES = 2 * N_TOK // TM + NUM_KEPT - 1
DUMMY_BLK = G_TILES
XS_ROWS = (G_TILES + 1) * TM

NW = 32
TPW = N_TOK // NW
CH = 16


def _router_body(x_ref, gw_ref, o2n_ref,
                 sel0_ref, sel1_ref, w0_ref, w1_ref, rank0_ref, rank1_ref,
                 offs_ref, tex_ref, tblk_ref, counts_scr):
    i = pl.program_id(0)

    @pl.when(i == 0)
    def _():
        counts_scr[...] = jnp.zeros_like(counts_scr)

    x = x_ref[...]
    gw = gw_ref[...]
    o2n = o2n_ref[...]
    tb = x.shape[0]

    logits = lax.dot_general(
        x, gw, (((1,), (1,)), ((), ())), preferred_element_type=jnp.float32
    )
    pruned = o2n < 0.0
    logits = jnp.where(pruned, NEG_INF, logits)

    m = jnp.max(logits, axis=-1, keepdims=True)
    ex = jnp.exp(logits - m)
    ex = jnp.where(pruned, 0.0, ex)
    p = ex / jnp.sum(ex, axis=-1, keepdims=True)

    iota = lax.broadcasted_iota(jnp.int32, (tb, NUM_EXPERTS), 1)
    big = jnp.int32(NUM_EXPERTS)
    v0 = jnp.max(p, axis=-1, keepdims=True)
    i0 = jnp.min(jnp.where(p == v0, iota, big), axis=-1, keepdims=True)
    p1m = jnp.where(iota == i0, -1.0, p)
    v1 = jnp.max(p1m, axis=-1, keepdims=True)
    i1 = jnp.min(jnp.where(p1m == v1, iota, big), axis=-1, keepdims=True)
    s = v0 + v1
    w0 = v0 / s
    w1 = v1 / s

    oh0_64 = (iota == i0).astype(jnp.float32)
    oh1_64 = (iota == i1).astype(jnp.float32)
    new0 = jnp.sum(oh0_64 * o2n, axis=-1, keepdims=True).astype(jnp.int32)
    new1 = jnp.sum(oh1_64 * o2n, axis=-1, keepdims=True).astype(jnp.int32)

    iota_k = lax.broadcasted_iota(jnp.int32, (tb, NUM_KEPT), 1)
    oh0 = (iota_k == new0).astype(jnp.float32)
    oh1 = (iota_k == new1).astype(jnp.float32)
    both = oh0 + oh1

    r_io = lax.broadcasted_iota(jnp.int32, (tb, tb), 0)
    c_io = lax.broadcasted_iota(jnp.int32, (tb, tb), 1)
    lt = (r_io > c_io).astype(jnp.float32)
    pre = lax.dot_general(
        lt, both, (((1,), (0,)), ((), ())), preferred_element_type=jnp.float32
    )

    counts = counts_scr[...]
    counts48 = counts[:, :NUM_KEPT]
    rank0 = jnp.sum((pre + counts48) * oh0, axis=-1, keepdims=True)
    rank1 = jnp.sum((pre + counts48) * oh1, axis=-1, keepdims=True)

    blk_tot = jnp.sum(both, axis=0, keepdims=True)
    pad = jnp.zeros((1, NUM_EXPERTS - NUM_KEPT), jnp.float32)
    counts_new = counts + jnp.concatenate([blk_tot, pad], axis=1)
    counts_scr[...] = counts_new

    tiles = jnp.floor((counts_new + (TM - 1.0)) * (1.0 / TM))
    a_io = lax.broadcasted_iota(jnp.int32, (NUM_EXPERTS, NUM_EXPERTS), 0)
    b_io = lax.broadcasted_iota(jnp.int32, (NUM_EXPERTS, NUM_EXPERTS), 1)
    excl = (a_io < b_io).astype(jnp.float32)
    tiles_excl = lax.dot_general(
        tiles, excl, (((1,), (0,)), ((), ())), preferred_element_type=jnp.float32
    )
    offs_ref[...] = (tiles_excl * float(TM)).astype(jnp.int32).reshape(
        1, 1, NUM_EXPERTS)

    n_tiles = jnp.sum(tiles, axis=-1, keepdims=True)
    tiles_incl = tiles_excl + tiles
    g_col = lax.broadcasted_iota(
        jnp.int32, (128, NUM_EXPERTS), 0).astype(jnp.float32)
    e_of_g = jnp.sum((tiles_incl <= g_col).astype(jnp.float32),
                     axis=-1, keepdims=True)
    g_row = g_col[:, 0:1]
    valid = g_row < n_tiles
    tile_ex = jnp.where(valid, jnp.minimum(e_of_g, float(NUM_KEPT - 1)),
                        float(NUM_KEPT - 1)).astype(jnp.int32)
    tile_blk = jnp.where(valid, g_row, float(DUMMY_BLK)).astype(jnp.int32)
    tex_ref[...] = tile_ex.reshape(1, 128, 1)
    tblk_ref[...] = tile_blk.reshape(1, 128, 1)

    sel0_ref[...] = new0.reshape(1, tb, 1)
    sel1_ref[...] = new1.reshape(1, tb, 1)
    w0_ref[...] = w0.reshape(1, tb, 1)
    w1_ref[...] = w1.reshape(1, tb, 1)
    rank0_ref[...] = rank0.astype(jnp.int32).reshape(1, tb, 1)
    rank1_ref[...] = rank1.astype(jnp.int32).reshape(1, tb, 1)


def _run_router(x, gate_weight, o2n_f):
    shp = functools.partial(jax.ShapeDtypeStruct)
    outs = pl.pallas_call(
        _router_body,
        grid=(NB_R,),
        in_specs=[
            pl.BlockSpec((TB_R, D_MODEL), lambda i: (i, 0)),
            pl.BlockSpec((NUM_EXPERTS, D_MODEL), lambda i: (0, 0)),
            pl.BlockSpec((1, NUM_EXPERTS), lambda i: (0, 0)),
        ],
        out_specs=[
            pl.BlockSpec((1, TB_R, 1), lambda i: (i, 0, 0)),
            pl.BlockSpec((1, TB_R, 1), lambda i: (i, 0, 0)),
            pl.BlockSpec((1, TB_R, 1), lambda i: (i, 0, 0)),
            pl.BlockSpec((1, TB_R, 1), lambda i: (i, 0, 0)),
            pl.BlockSpec((1, TB_R, 1), lambda i: (i, 0, 0)),
            pl.BlockSpec((1, TB_R, 1), lambda i: (i, 0, 0)),
            pl.BlockSpec((1, 1, NUM_EXPERTS), lambda i: (0, 0, 0)),
            pl.BlockSpec((1, 128, 1), lambda i: (0, 0, 0)),
            pl.BlockSpec((1, 128, 1), lambda i: (0, 0, 0)),
        ],
        out_shape=[
            shp((NB_R, TB_R, 1), jnp.int32),
            shp((NB_R, TB_R, 1), jnp.int32),
            shp((NB_R, TB_R, 1), jnp.float32),
            shp((NB_R, TB_R, 1), jnp.float32),
            shp((NB_R, TB_R, 1), jnp.int32),
            shp((NB_R, TB_R, 1), jnp.int32),
            shp((1, 1, NUM_EXPERTS), jnp.int32),
            shp((1, 128, 1), jnp.int32),
            shp((1, 128, 1), jnp.int32),
        ],
        scratch_shapes=[pltpu.VMEM((1, NUM_EXPERTS), jnp.float32)],
    )(x, gate_weight, o2n_f)
    sel0, sel1, w0, w1, rank0, rank1, offs, tex, tblk = outs
    flat = lambda a: a.reshape(N_TOK)
    return (flat(sel0), flat(sel1), flat(w0), flat(w1),
            flat(rank0), flat(rank1), offs.reshape(NUM_EXPERTS),
            tex.reshape(128), tblk.reshape(128))


def _compute_pos(offs_v, sel_v, rank_v, pos_v):
    tabs = [offs_v[pl.ds(j * 16, 16)] for j in range(4)]
    for c in range(TPW // CH):
        sl = pl.ds(c * CH, CH)
        s = sel_v[sl]
        k = jnp.bitwise_and(s, 15)
        j = jnp.right_shift(s, 4)
        acc = tabs[0].at[k].get(mode="promise_in_bounds")
        for jc in range(1, 4):
            acc = jnp.where(j == jc,
                            tabs[jc].at[k].get(mode="promise_in_bounds"),
                            acc)
        pos_v[sl] = acc + rank_v[sl]


def _sc_scatter_body(x_hbm, sel0_hbm, sel1_hbm, rank0_hbm, rank1_hbm, offs_hbm,
                     xs_hbm, pos0_hbm, pos1_hbm,
                     offs_v, sel_v, rank_v, pos0_v, pos1_v,
                     xrow_a, xrow_b, sra, srb, swa, swb):
    cid = lax.axis_index("c")
    sid = lax.axis_index("s")
    wid = sid * 2 + cid
    base = wid * TPW

    pltpu.sync_copy(offs_hbm, offs_v)
    pltpu.sync_copy(sel0_hbm.at[pl.ds(base, TPW)], sel_v)
    pltpu.sync_copy(rank0_hbm.at[pl.ds(base, TPW)], rank_v)
    _compute_pos(offs_v, sel_v, rank_v, pos0_v)
    pltpu.sync_copy(sel1_hbm.at[pl.ds(base, TPW)], sel_v)
    pltpu.sync_copy(rank1_hbm.at[pl.ds(base, TPW)], rank_v)
    _compute_pos(offs_v, sel_v, rank_v, pos1_v)
    pltpu.sync_copy(pos0_v, pos0_hbm.at[pl.ds(base, TPW)])
    pltpu.sync_copy(pos1_v, pos1_hbm.at[pl.ds(base, TPW)])

    nch = TPW // CH
    bufs = (xrow_a, xrow_b)
    rsems = (sra, srb)
    wsems = (swa, swb)

    def rd(c, buf, sem):
        return pltpu.make_async_copy(
            x_hbm.at[pl.ds(base + c * CH, CH)], buf, sem)

    def wr(c, buf, sem):
        sl = pl.ds(c * CH, CH)
        return (pltpu.make_async_copy(buf, xs_hbm.at[pos0_v[sl]], sem),
                pltpu.make_async_copy(buf, xs_hbm.at[pos1_v[sl]], sem))

    rd(0, bufs[0], rsems[0]).start()
    for c in range(nch):
        p = c % 2
        if c + 1 < nch:
            if c >= 1:
                w0d, w1d = wr(c - 1, bufs[1 - p], wsems[1 - p])
                w0d.wait()
                w1d.wait()
            rd(c + 1, bufs[1 - p], rsems[1 - p]).start()
        rd(c, bufs[p], rsems[p]).wait()
        w0d, w1d = wr(c, bufs[p], wsems[p])
        w0d.start()
        w1d.start()
    for c in (nch - 2, nch - 1):
        p = c % 2
        w0d, w1d = wr(c, bufs[p], wsems[p])
        w0d.wait()
        w1d.wait()


def _run_scatter(x, sel0, sel1, rank0, rank1, offs):
    mesh = plsc.VectorSubcoreMesh(core_axis_name="c", subcore_axis_name="s")
    shp = jax.ShapeDtypeStruct
    f = pl.kernel(
        _sc_scatter_body,
        out_type=[
            shp((XS_ROWS, D_MODEL), jnp.float32),
            shp((N_TOK,), jnp.int32),
            shp((N_TOK,), jnp.int32),
        ],
        mesh=mesh,
        scratch_types=[
            pltpu.VMEM((NUM_EXPERTS,), jnp.int32),
            pltpu.VMEM((TPW,), jnp.int32),
            pltpu.VMEM((TPW,), jnp.int32),
            pltpu.VMEM((TPW,), jnp.int32),
            pltpu.VMEM((TPW,), jnp.int32),
            pltpu.VMEM((CH, D_MODEL), jnp.float32),
            pltpu.VMEM((CH, D_MODEL), jnp.float32),
            pltpu.SemaphoreType.DMA,
            pltpu.SemaphoreType.DMA,
            pltpu.SemaphoreType.DMA,
            pltpu.SemaphoreType.DMA,
        ],
    )
    return f(x, sel0, sel1, rank0, rank1, offs)


def _gemm_body(tex_smem, tblk_smem, x_ref, guw_ref, dpw_ref, y_ref):
    g_id = pl.program_id(0)

    @pl.when(tblk_smem[g_id] != DUMMY_BLK)
    def _():
        xt = x_ref[...]
        guw = guw_ref[0]
        dpw = dpw_ref[0]
        gu = lax.dot_general(
            xt, guw, (((1,), (1,)), ((), ())),
            preferred_element_type=jnp.float32
        )
        g = gu[:, :D_FF]
        u = gu[:, D_FF:]
        h = g * lax.logistic(g) * u
        y_ref[...] = lax.dot_general(
            h, dpw, (((1,), (1,)), ((), ())),
            preferred_element_type=jnp.float32
        ).astype(jnp.bfloat16)


def _run_gemm(tex, tblk, xs, gate_up_proj, down_proj):
    grid_spec = pltpu.PrefetchScalarGridSpec(
        num_scalar_prefetch=2,
        grid=(G_TILES,),
        in_specs=[
            pl.BlockSpec((TM, D_MODEL), lambda g, tex, tblk: (tblk[g], 0)),
            pl.BlockSpec((1, TWO_F, D_MODEL), lambda g, tex, tblk: (tex[g], 0, 0)),
            pl.BlockSpec((1, D_MODEL, D_FF), lambda g, tex, tblk: (tex[g], 0, 0)),
        ],
        out_specs=pl.BlockSpec((TM, D_MODEL), lambda g, tex, tblk: (tblk[g], 0)),
    )
    return pl.pallas_call(
        _gemm_body,
        grid_spec=grid_spec,
        out_shape=jax.ShapeDtypeStruct((XS_ROWS, D_MODEL), jnp.bfloat16),
    )(tex, tblk, xs, gate_up_proj, down_proj)


def _sc_combine_body(ys_hbm, pos0_hbm, pos1_hbm, w0_hbm, w1_hbm,
                     out_hbm,
                     pos0_v, pos1_v, w0_v, w1_v,
                     rows0_a, rows0_b, rows1_a, rows1_b, out_a, out_b,
                     sem0a, sem0b, sem1a, sem1b, osema, osemb):
    cid = lax.axis_index("c")
    sid = lax.axis_index("s")
    wid = sid * 2 + cid
    base = wid * TPW

    pltpu.sync_copy(pos0_hbm.at[pl.ds(base, TPW)], pos0_v)
    pltpu.sync_copy(pos1_hbm.at[pl.ds(base, TPW)], pos1_v)
    pltpu.sync_copy(w0_hbm.at[pl.ds(base, TPW)], w0_v)
    pltpu.sync_copy(w1_hbm.at[pl.ds(base, TPW)], w1_v)

    nch = TPW // CH
    r0 = (rows0_a, rows0_b)
    r1 = (rows1_a, rows1_b)
    sems0 = (sem0a, sem0b)
    sems1 = (sem1a, sem1b)
    osems = (osema, osemb)
    outs = (out_a, out_b)

    def gathers(c, p):
        sl = pl.ds(c * CH, CH)
        return (pltpu.make_async_copy(ys_hbm.at[pos0_v[sl]], r0[p], sems0[p]),
                pltpu.make_async_copy(ys_hbm.at[pos1_v[sl]], r1[p], sems1[p]))

    g0, g1 = gathers(0, 0)
    g0.start()
    g1.start()
    for c in range(nch):
        p = c % 2
        if c + 1 < nch:
            g0, g1 = gathers(c + 1, 1 - p)
            g0.start()
            g1.start()
        g0, g1 = gathers(c, p)
        g0.wait()
        g1.wait()
        if c >= 2:
            pltpu.make_async_copy(
                outs[p], out_hbm.at[pl.ds(base + (c - 2) * CH, CH)],
                osems[p]).wait()
        sl = pl.ds(c * CH, CH)
        wv0 = w0_v[sl]
        wv1 = w1_v[sl]
        for r in range(CH):
            a = wv0[r]
            b = wv1[r]

            def dbody(d, _):
                for j in range(2):
                    dsl = pl.ds(d * 32 + j * 16, 16)
                    v0 = plsc.bitcast(r0[p][r, dsl], jnp.bfloat16)
                    v1 = plsc.bitcast(r1[p][r, dsl], jnp.bfloat16)
                    a0, a1 = plsc.unpack(v0, format=plsc.PackFormat.INTERLEAVED)
                    b0, b1 = plsc.unpack(v1, format=plsc.PackFormat.INTERLEAVED)
                    o0 = a0 * a + b0 * b
                    o1 = a1 * a + b1 * b
                    o = plsc.pack(o0, o1, format=plsc.PackFormat.INTERLEAVED)
                    outs[p][r, dsl] = plsc.bitcast(o, jnp.int32)
                return 0

            lax.fori_loop(0, D_HALF // 32, dbody, 0)
        pltpu.make_async_copy(
            outs[p], out_hbm.at[pl.ds(base + c * CH, CH)], osems[p]).start()
    for c in (nch - 2, nch - 1):
        p = c % 2
        pltpu.make_async_copy(
            outs[p], out_hbm.at[pl.ds(base + c * CH, CH)], osems[p]).wait()


def _run_combine(ys_i, pos0, pos1, w0, w1):
    mesh = plsc.VectorSubcoreMesh(core_axis_name="c", subcore_axis_name="s")
    f = pl.kernel(
        _sc_combine_body,
        out_type=jax.ShapeDtypeStruct((N_TOK, D_HALF), jnp.int32),
        mesh=mesh,
        compiler_params=pltpu.CompilerParams(needs_layout_passes=False),
        scratch_types=[
            pltpu.VMEM((TPW,), jnp.int32),
            pltpu.VMEM((TPW,), jnp.int32),
            pltpu.VMEM((TPW,), jnp.float32),
            pltpu.VMEM((TPW,), jnp.float32),
            pltpu.VMEM((CH, D_HALF), jnp.int32),
            pltpu.VMEM((CH, D_HALF), jnp.int32),
            pltpu.VMEM((CH, D_HALF), jnp.int32),
            pltpu.VMEM((CH, D_HALF), jnp.int32),
            pltpu.VMEM((CH, D_HALF), jnp.int32),
            pltpu.VMEM((CH, D_HALF), jnp.int32),
            pltpu.SemaphoreType.DMA,
            pltpu.SemaphoreType.DMA,
            pltpu.SemaphoreType.DMA,
            pltpu.SemaphoreType.DMA,
            pltpu.SemaphoreType.DMA,
            pltpu.SemaphoreType.DMA,
        ],
    )
    return f(ys_i, pos0, pos1, w0, w1)


@jax.jit
def kernel(hidden_states, gate_weight, gate_up_proj, down_proj, old_to_new):
    bsz, seq, dim = hidden_states.shape
    x = hidden_states.reshape(bsz * seq, dim)
    o2n_f = old_to_new.astype(jnp.float32).reshape(1, NUM_EXPERTS)

    (sel0, sel1, w0, w1, rank0, rank1, offs, tex, tblk) = _run_router(
        x, gate_weight, o2n_f)
    xs, pos0, pos1 = _run_scatter(x, sel0, sel1, rank0, rank1, offs)
    ys = _run_gemm(tex, tblk, xs, gate_up_proj, down_proj)
    ys_i = lax.bitcast_convert_type(
        ys.reshape(XS_ROWS, D_HALF, 2), jnp.int32)
    out_i = _run_combine(ys_i, pos0, pos1, w0, w1)
    out = lax.bitcast_convert_type(out_i, jnp.bfloat16).reshape(
        bsz * seq, D_MODEL)
    return out.astype(jnp.float32).reshape(bsz, seq, dim)


# speedup vs baseline: 3.3291x; 3.3291x over previous
import functools

import jax
import jax.numpy as jnp
from jax import lax
from jax.experimental import pallas as pl
from jax.experimental.pallas import tpu as pltpu
from jax.experimental.pallas import tpu_sc as plsc

NUM_EXPERTS = 64
NUM_KEPT = 48
NEG_INF = float("-inf")

D_MODEL = 1024
D_HALF = D_MODEL // 2
D_FF = 384
TWO_F = 2 * D_FF
N_TOK = 4096
TB_R = 512
NB_R = N_TOK // TB_R
TM = 256
G_TILES = 2 * N_TOK // TM + NUM_KEPT - 1
DUMMY_BLK = G_TILES
XS_ROWS = (G_TILES + 1) * TM

NW = 32
TPW = N_TOK // NW
CH = 16


def _router_body(x_ref, gw_ref, o2n_ref,
                 sel0_ref, sel1_ref, w0_ref, w1_ref, rank0_ref, rank1_ref,
                 offs_ref, tex_ref, tblk_ref, counts_scr):
    i = pl.program_id(0)

    @pl.when(i == 0)
    def _():
        counts_scr[...] = jnp.zeros_like(counts_scr)

    x = x_ref[...]
    gw = gw_ref[...]
    o2n = o2n_ref[...]
    tb = x.shape[0]

    logits = lax.dot_general(
        x, gw, (((1,), (1,)), ((), ())), preferred_element_type=jnp.float32
    )
    pruned = o2n < 0.0
    logits = jnp.where(pruned, NEG_INF, logits)

    m = jnp.max(logits, axis=-1, keepdims=True)
    ex = jnp.exp(logits - m)
    ex = jnp.where(pruned, 0.0, ex)
    p = ex / jnp.sum(ex, axis=-1, keepdims=True)

    iota = lax.broadcasted_iota(jnp.int32, (tb, NUM_EXPERTS), 1)
    big = jnp.int32(NUM_EXPERTS)
    v0 = jnp.max(p, axis=-1, keepdims=True)
    i0 = jnp.min(jnp.where(p == v0, iota, big), axis=-1, keepdims=True)
    p1m = jnp.where(iota == i0, -1.0, p)
    v1 = jnp.max(p1m, axis=-1, keepdims=True)
    i1 = jnp.min(jnp.where(p1m == v1, iota, big), axis=-1, keepdims=True)
    s = v0 + v1
    w0 = v0 / s
    w1 = v1 / s

    oh0_64 = (iota == i0).astype(jnp.float32)
    oh1_64 = (iota == i1).astype(jnp.float32)
    new0 = jnp.sum(oh0_64 * o2n, axis=-1, keepdims=True).astype(jnp.int32)
    new1 = jnp.sum(oh1_64 * o2n, axis=-1, keepdims=True).astype(jnp.int32)

    iota_k = lax.broadcasted_iota(jnp.int32, (tb, NUM_KEPT), 1)
    oh0 = (iota_k == new0).astype(jnp.float32)
    oh1 = (iota_k == new1).astype(jnp.float32)
    both = oh0 + oh1

    r_io = lax.broadcasted_iota(jnp.int32, (tb, tb), 0)
    c_io = lax.broadcasted_iota(jnp.int32, (tb, tb), 1)
    lt = (r_io > c_io).astype(jnp.float32)
    pre = lax.dot_general(
        lt, both, (((1,), (0,)), ((), ())), preferred_element_type=jnp.float32
    )

    counts = counts_scr[...]
    counts48 = counts[:, :NUM_KEPT]
    rank0 = jnp.sum((pre + counts48) * oh0, axis=-1, keepdims=True)
    rank1 = jnp.sum((pre + counts48) * oh1, axis=-1, keepdims=True)

    blk_tot = jnp.sum(both, axis=0, keepdims=True)
    pad = jnp.zeros((1, NUM_EXPERTS - NUM_KEPT), jnp.float32)
    counts_new = counts + jnp.concatenate([blk_tot, pad], axis=1)
    counts_scr[...] = counts_new

    tiles = jnp.floor((counts_new + (TM - 1.0)) * (1.0 / TM))
    a_io = lax.broadcasted_iota(jnp.int32, (NUM_EXPERTS, NUM_EXPERTS), 0)
    b_io = lax.broadcasted_iota(jnp.int32, (NUM_EXPERTS, NUM_EXPERTS), 1)
    excl = (a_io < b_io).astype(jnp.float32)
    tiles_excl = lax.dot_general(
        tiles, excl, (((1,), (0,)), ((), ())), preferred_element_type=jnp.float32
    )
    offs_ref[...] = (tiles_excl * float(TM)).astype(jnp.int32).reshape(
        1, 1, NUM_EXPERTS)

    n_tiles = jnp.sum(tiles, axis=-1, keepdims=True)
    tiles_incl = tiles_excl + tiles
    g_col = lax.broadcasted_iota(
        jnp.int32, (128, NUM_EXPERTS), 0).astype(jnp.float32)
    e_of_g = jnp.sum((tiles_incl <= g_col).astype(jnp.float32),
                     axis=-1, keepdims=True)
    g_row = g_col[:, 0:1]
    valid = g_row < n_tiles
    tile_ex = jnp.where(valid, jnp.minimum(e_of_g, float(NUM_KEPT - 1)),
                        float(NUM_KEPT - 1)).astype(jnp.int32)
    tile_blk = jnp.where(valid, g_row, float(DUMMY_BLK)).astype(jnp.int32)
    tex_ref[...] = tile_ex.reshape(1, 128, 1)
    tblk_ref[...] = tile_blk.reshape(1, 128, 1)

    sel0_ref[...] = new0.reshape(1, tb, 1)
    sel1_ref[...] = new1.reshape(1, tb, 1)
    w0_ref[...] = w0.reshape(1, tb, 1)
    w1_ref[...] = w1.reshape(1, tb, 1)
    rank0_ref[...] = rank0.astype(jnp.int32).reshape(1, tb, 1)
    rank1_ref[...] = rank1.astype(jnp.int32).reshape(1, tb, 1)


def _run_router(x, gate_weight, o2n_f):
    shp = functools.partial(jax.ShapeDtypeStruct)
    outs = pl.pallas_call(
        _router_body,
        grid=(NB_R,),
        in_specs=[
            pl.BlockSpec((TB_R, D_MODEL), lambda i: (i, 0)),
            pl.BlockSpec((NUM_EXPERTS, D_MODEL), lambda i: (0, 0)),
            pl.BlockSpec((1, NUM_EXPERTS), lambda i: (0, 0)),
        ],
        out_specs=[
            pl.BlockSpec((1, TB_R, 1), lambda i: (i, 0, 0)),
            pl.BlockSpec((1, TB_R, 1), lambda i: (i, 0, 0)),
            pl.BlockSpec((1, TB_R, 1), lambda i: (i, 0, 0)),
            pl.BlockSpec((1, TB_R, 1), lambda i: (i, 0, 0)),
            pl.BlockSpec((1, TB_R, 1), lambda i: (i, 0, 0)),
            pl.BlockSpec((1, TB_R, 1), lambda i: (i, 0, 0)),
            pl.BlockSpec((1, 1, NUM_EXPERTS), lambda i: (0, 0, 0)),
            pl.BlockSpec((1, 128, 1), lambda i: (0, 0, 0)),
            pl.BlockSpec((1, 128, 1), lambda i: (0, 0, 0)),
        ],
        out_shape=[
            shp((NB_R, TB_R, 1), jnp.int32),
            shp((NB_R, TB_R, 1), jnp.int32),
            shp((NB_R, TB_R, 1), jnp.float32),
            shp((NB_R, TB_R, 1), jnp.float32),
            shp((NB_R, TB_R, 1), jnp.int32),
            shp((NB_R, TB_R, 1), jnp.int32),
            shp((1, 1, NUM_EXPERTS), jnp.int32),
            shp((1, 128, 1), jnp.int32),
            shp((1, 128, 1), jnp.int32),
        ],
        scratch_shapes=[pltpu.VMEM((1, NUM_EXPERTS), jnp.float32)],
    )(x, gate_weight, o2n_f)
    sel0, sel1, w0, w1, rank0, rank1, offs, tex, tblk = outs
    flat = lambda a: a.reshape(N_TOK)
    return (flat(sel0), flat(sel1), flat(w0), flat(w1),
            flat(rank0), flat(rank1), offs.reshape(NUM_EXPERTS),
            tex.reshape(128), tblk.reshape(128))


def _compute_pos(offs_v, sel_v, rank_v, pos_v):
    tabs = [offs_v[pl.ds(j * 16, 16)] for j in range(4)]
    for c in range(TPW // CH):
        sl = pl.ds(c * CH, CH)
        s = sel_v[sl]
        k = jnp.bitwise_and(s, 15)
        j = jnp.right_shift(s, 4)
        acc = tabs[0].at[k].get(mode="promise_in_bounds")
        for jc in range(1, 4):
            acc = jnp.where(j == jc,
                            tabs[jc].at[k].get(mode="promise_in_bounds"),
                            acc)
        pos_v[sl] = acc + rank_v[sl]


def _sc_scatter_body(x_hbm, sel0_hbm, sel1_hbm, rank0_hbm, rank1_hbm, offs_hbm,
                     xs_hbm, pos0_hbm, pos1_hbm,
                     offs_v, sel_v, rank_v, pos0_v, pos1_v,
                     xrow_a, xrow_b, sra, srb, swa, swb):
    cid = lax.axis_index("c")
    sid = lax.axis_index("s")
    wid = sid * 2 + cid
    base = wid * TPW

    pltpu.sync_copy(offs_hbm, offs_v)
    pltpu.sync_copy(sel0_hbm.at[pl.ds(base, TPW)], sel_v)
    pltpu.sync_copy(rank0_hbm.at[pl.ds(base, TPW)], rank_v)
    _compute_pos(offs_v, sel_v, rank_v, pos0_v)
    pltpu.sync_copy(sel1_hbm.at[pl.ds(base, TPW)], sel_v)
    pltpu.sync_copy(rank1_hbm.at[pl.ds(base, TPW)], rank_v)
    _compute_pos(offs_v, sel_v, rank_v, pos1_v)
    pltpu.sync_copy(pos0_v, pos0_hbm.at[pl.ds(base, TPW)])
    pltpu.sync_copy(pos1_v, pos1_hbm.at[pl.ds(base, TPW)])

    nch = TPW // CH
    bufs = (xrow_a, xrow_b)
    rsems = (sra, srb)
    wsems = (swa, swb)

    def rd(c, buf, sem):
        return pltpu.make_async_copy(
            x_hbm.at[pl.ds(base + c * CH, CH)], buf, sem)

    def wr(c, buf, sem):
        sl = pl.ds(c * CH, CH)
        return (pltpu.make_async_copy(buf, xs_hbm.at[pos0_v[sl]], sem),
                pltpu.make_async_copy(buf, xs_hbm.at[pos1_v[sl]], sem))

    rd(0, bufs[0], rsems[0]).start()
    for c in range(nch):
        p = c % 2
        if c + 1 < nch:
            if c >= 1:
                w0d, w1d = wr(c - 1, bufs[1 - p], wsems[1 - p])
                w0d.wait()
                w1d.wait()
            rd(c + 1, bufs[1 - p], rsems[1 - p]).start()
        rd(c, bufs[p], rsems[p]).wait()
        w0d, w1d = wr(c, bufs[p], wsems[p])
        w0d.start()
        w1d.start()
    for c in (nch - 2, nch - 1):
        p = c % 2
        w0d, w1d = wr(c, bufs[p], wsems[p])
        w0d.wait()
        w1d.wait()


def _run_scatter(x, sel0, sel1, rank0, rank1, offs):
    mesh = plsc.VectorSubcoreMesh(core_axis_name="c", subcore_axis_name="s")
    shp = jax.ShapeDtypeStruct
    f = pl.kernel(
        _sc_scatter_body,
        out_type=[
            shp((XS_ROWS, D_MODEL), jnp.float32),
            shp((N_TOK,), jnp.int32),
            shp((N_TOK,), jnp.int32),
        ],
        mesh=mesh,
        scratch_types=[
            pltpu.VMEM((NUM_EXPERTS,), jnp.int32),
            pltpu.VMEM((TPW,), jnp.int32),
            pltpu.VMEM((TPW,), jnp.int32),
            pltpu.VMEM((TPW,), jnp.int32),
            pltpu.VMEM((TPW,), jnp.int32),
            pltpu.VMEM((CH, D_MODEL), jnp.float32),
            pltpu.VMEM((CH, D_MODEL), jnp.float32),
            pltpu.SemaphoreType.DMA,
            pltpu.SemaphoreType.DMA,
            pltpu.SemaphoreType.DMA,
            pltpu.SemaphoreType.DMA,
        ],
    )
    return f(x, sel0, sel1, rank0, rank1, offs)


def _gemm_body(tex_smem, tblk_smem, x_ref, guw_ref, dpw_ref, y_ref):
    g_id = pl.program_id(0)

    @pl.when(tblk_smem[g_id] != DUMMY_BLK)
    def _():
        xt = x_ref[...]
        guw = guw_ref[0]
        dpw = dpw_ref[0]
        gu = lax.dot_general(
            xt, guw, (((1,), (1,)), ((), ())),
            preferred_element_type=jnp.float32
        )
        g = gu[:, :D_FF]
        u = gu[:, D_FF:]
        h = g * lax.logistic(g) * u
        y_ref[...] = lax.dot_general(
            h, dpw, (((1,), (1,)), ((), ())),
            preferred_element_type=jnp.float32
        )


def _run_gemm(tex, tblk, xs, gate_up_proj, down_proj):
    grid_spec = pltpu.PrefetchScalarGridSpec(
        num_scalar_prefetch=2,
        grid=(G_TILES,),
        in_specs=[
            pl.BlockSpec((TM, D_MODEL), lambda g, tex, tblk: (tblk[g], 0)),
            pl.BlockSpec((1, TWO_F, D_MODEL), lambda g, tex, tblk: (tex[g], 0, 0)),
            pl.BlockSpec((1, D_MODEL, D_FF), lambda g, tex, tblk: (tex[g], 0, 0)),
        ],
        out_specs=pl.BlockSpec((TM, D_MODEL), lambda g, tex, tblk: (tblk[g], 0)),
    )
    return pl.pallas_call(
        _gemm_body,
        grid_spec=grid_spec,
        out_shape=jax.ShapeDtypeStruct((XS_ROWS, D_MODEL), jnp.float32),
    )(tex, tblk, xs, gate_up_proj, down_proj)


def _sc_combine_body(ys_hbm, pos0_hbm, pos1_hbm, w0_hbm, w1_hbm,
                     out_hbm,
                     pos0_v, pos1_v, w0_v, w1_v,
                     rows0_a, rows0_b, rows1_a, rows1_b, out_a, out_b,
                     sem0a, sem0b, sem1a, sem1b, osema, osemb):
    cid = lax.axis_index("c")
    sid = lax.axis_index("s")
    wid = sid * 2 + cid
    base = wid * TPW

    pltpu.sync_copy(pos0_hbm.at[pl.ds(base, TPW)], pos0_v)
    pltpu.sync_copy(pos1_hbm.at[pl.ds(base, TPW)], pos1_v)
    pltpu.sync_copy(w0_hbm.at[pl.ds(base, TPW)], w0_v)
    pltpu.sync_copy(w1_hbm.at[pl.ds(base, TPW)], w1_v)

    nch = TPW // CH
    r0 = (rows0_a, rows0_b)
    r1 = (rows1_a, rows1_b)
    sems0 = (sem0a, sem0b)
    sems1 = (sem1a, sem1b)
    osems = (osema, osemb)
    outs = (out_a, out_b)

    def gathers(c, p):
        sl = pl.ds(c * CH, CH)
        return (pltpu.make_async_copy(ys_hbm.at[pos0_v[sl]], r0[p], sems0[p]),
                pltpu.make_async_copy(ys_hbm.at[pos1_v[sl]], r1[p], sems1[p]))

    g0, g1 = gathers(0, 0)
    g0.start()
    g1.start()
    for c in range(nch):
        p = c % 2
        if c + 1 < nch:
            g0, g1 = gathers(c + 1, 1 - p)
            g0.start()
            g1.start()
        g0, g1 = gathers(c, p)
        g0.wait()
        g1.wait()
        if c >= 2:
            pltpu.make_async_copy(
                outs[p], out_hbm.at[pl.ds(base + (c - 2) * CH, CH)],
                osems[p]).wait()
        sl = pl.ds(c * CH, CH)
        wv0 = w0_v[sl]
        wv1 = w1_v[sl]
        for r in range(CH):
            a = wv0[r]
            b = wv1[r]

            def dbody(d, _):
                for j in range(4):
                    dsl = pl.ds(d * 64 + j * 16, 16)
                    outs[p][r, dsl] = r0[p][r, dsl] * a + r1[p][r, dsl] * b
                return 0

            lax.fori_loop(0, D_MODEL // 64, dbody, 0)
        pltpu.make_async_copy(
            outs[p], out_hbm.at[pl.ds(base + c * CH, CH)], osems[p]).start()
    for c in (nch - 2, nch - 1):
        p = c % 2
        pltpu.make_async_copy(
            outs[p], out_hbm.at[pl.ds(base + c * CH, CH)], osems[p]).wait()


def _run_combine(ys, pos0, pos1, w0, w1):
    mesh = plsc.VectorSubcoreMesh(core_axis_name="c", subcore_axis_name="s")
    f = pl.kernel(
        _sc_combine_body,
        out_type=jax.ShapeDtypeStruct((N_TOK, D_MODEL), jnp.float32),
        mesh=mesh,
        scratch_types=[
            pltpu.VMEM((TPW,), jnp.int32),
            pltpu.VMEM((TPW,), jnp.int32),
            pltpu.VMEM((TPW,), jnp.float32),
            pltpu.VMEM((TPW,), jnp.float32),
            pltpu.VMEM((CH, D_MODEL), jnp.float32),
            pltpu.VMEM((CH, D_MODEL), jnp.float32),
            pltpu.VMEM((CH, D_MODEL), jnp.float32),
            pltpu.VMEM((CH, D_MODEL), jnp.float32),
            pltpu.VMEM((CH, D_MODEL), jnp.float32),
            pltpu.VMEM((CH, D_MODEL), jnp.float32),
            pltpu.SemaphoreType.DMA,
            pltpu.SemaphoreType.DMA,
            pltpu.SemaphoreType.DMA,
            pltpu.SemaphoreType.DMA,
            pltpu.SemaphoreType.DMA,
            pltpu.SemaphoreType.DMA,
        ],
    )
    return f(ys, pos0, pos1, w0, w1)


@jax.jit
def kernel(hidden_states, gate_weight, gate_up_proj, down_proj, old_to_new):
    bsz, seq, dim = hidden_states.shape
    x = hidden_states.reshape(bsz * seq, dim)
    o2n_f = old_to_new.astype(jnp.float32).reshape(1, NUM_EXPERTS)

    (sel0, sel1, w0, w1, rank0, rank1, offs, tex, tblk) = _run_router(
        x, gate_weight, o2n_f)
    xs, pos0, pos1 = _run_scatter(x, sel0, sel1, rank0, rank1, offs)
    ys = _run_gemm(tex, tblk, xs, gate_up_proj, down_proj)
    out = _run_combine(ys, pos0, pos1, w0, w1)
    return out.reshape(bsz, seq, dim)
